# baseline (speedup 1.0000x reference)
import jax
import jax.numpy as jnp
from jax.experimental import pallas as pl
from jax.experimental.pallas import tpu as pltpu

_L = 128
_BG = 8


def _ssm_body(u_ref, h0_ref, logA_ref, logdt_ref, Bb_ref, BwT_ref, CwT_ref,
              Cb_ref, Dp_ref, y_ref, h_ref):
    t = pl.program_id(1)
    N = logA_ref.shape[-1]
    L = u_ref.shape[1]
    bg = u_ref.shape[0]
    D = u_ref.shape[2]

    dt = jnp.minimum(jnp.exp(logdt_ref[...]), 1.0)
    a_bar = jnp.exp(-jnp.exp(logA_ref[...]) * dt)
    bias = Bb_ref[...] * dt

    @pl.when(t == 0)
    def _():
        h_ref[0:bg, :] = h0_ref[0]

    u3 = u_ref[...]
    u2 = u3.reshape(bg * L, D)
    bu = jnp.dot(u2, BwT_ref[...], preferred_element_type=jnp.float32)
    bu = bu * dt + bias

    for b in range(bg):
        s = bu[b * L:(b + 1) * L, :]
        first = s[0:1, :] + a_bar * h_ref[b:b + 1, :]
        s = jnp.concatenate([first, s[1:, :]], axis=0)
        p = a_bar
        d = 1
        while d < L:
            shifted = jnp.concatenate(
                [jnp.zeros((d, N), jnp.float32), s[:L - d, :]], axis=0)
            s = s + p * shifted
            p = p * p
            d *= 2
        h_ref[b:b + 1, :] = s[L - 1:L, :]
        y = jnp.dot(s, CwT_ref[...], preferred_element_type=jnp.float32)
        y_ref[b] = y + Cb_ref[...] + Dp_ref[...] * u3[b]


def kernel(u, h0, log_A, B_w, B_b, C_w, C_b, D_param, log_dt):
    B, T, D = u.shape
    N = log_A.shape[0]
    bg = min(_BG, B)
    L = min(_L, T)
    n_g = B // bg
    grid = (n_g, T // L)
    h0r = h0.reshape(n_g, bg, N)
    return pl.pallas_call(
        _ssm_body,
        grid=grid,
        in_specs=[
            pl.BlockSpec((bg, L, D), lambda g, t: (g, t, 0)),
            pl.BlockSpec((1, bg, N), lambda g, t: (g, 0, 0)),
            pl.BlockSpec((1, N), lambda g, t: (0, 0)),
            pl.BlockSpec((1, N), lambda g, t: (0, 0)),
            pl.BlockSpec((1, N), lambda g, t: (0, 0)),
            pl.BlockSpec((D, N), lambda g, t: (0, 0)),
            pl.BlockSpec((N, D), lambda g, t: (0, 0)),
            pl.BlockSpec((1, D), lambda g, t: (0, 0)),
            pl.BlockSpec((1, D), lambda g, t: (0, 0)),
        ],
        out_specs=pl.BlockSpec((bg, L, D), lambda g, t: (g, t, 0)),
        out_shape=jax.ShapeDtypeStruct((B, T, D), jnp.float32),
        scratch_shapes=[pltpu.VMEM((8, N), jnp.float32)],
        compiler_params=pltpu.CompilerParams(
            dimension_semantics=("parallel", "arbitrary"),
            vmem_limit_bytes=56 * 1024 * 1024,
        ),
        name="ssm_layer",
    )(u, h0r, log_A.reshape(1, N), log_dt.reshape(1, N), B_b.reshape(1, N),
      B_w.T, C_w.T, C_b.reshape(1, D), D_param.reshape(1, D))


# speedup vs baseline: 1.0837x; 1.0837x over previous
import jax
import jax.numpy as jnp
from jax.experimental import pallas as pl
from jax.experimental.pallas import tpu as pltpu

_L = 256
_BG = 8


def _ssm_body(u_ref, h0_ref, logA_ref, logdt_ref, Bb_ref, BwT_ref, CwT_ref,
              Cb_ref, Dp_ref, y_ref, h_ref):
    t = pl.program_id(1)
    N = logA_ref.shape[-1]
    L = u_ref.shape[1]
    bg = u_ref.shape[0]
    D = u_ref.shape[2]

    dt = jnp.minimum(jnp.exp(logdt_ref[...]), 1.0)
    a_bar = jnp.exp(-jnp.exp(logA_ref[...]) * dt)
    bias = Bb_ref[...] * dt

    @pl.when(t == 0)
    def _():
        h_ref[0:bg, :] = h0_ref[0]

    u3 = u_ref[...]
    u2 = u3.reshape(bg * L, D)
    bu = jnp.dot(u2, BwT_ref[...], preferred_element_type=jnp.float32)
    bu = bu * dt + bias

    for b in range(bg):
        s = bu[b * L:(b + 1) * L, :]
        first = s[0:1, :] + a_bar * h_ref[b:b + 1, :]
        s = jnp.concatenate([first, s[1:, :]], axis=0)
        p = a_bar
        d = 1
        while d < L:
            shifted = jnp.concatenate(
                [jnp.zeros((d, N), jnp.float32), s[:L - d, :]], axis=0)
            s = s + p * shifted
            p = p * p
            d *= 2
        h_ref[b:b + 1, :] = s[L - 1:L, :]
        y = jnp.dot(s, CwT_ref[...], preferred_element_type=jnp.float32)
        y_ref[b] = y + Cb_ref[...] + Dp_ref[...] * u3[b]


def kernel(u, h0, log_A, B_w, B_b, C_w, C_b, D_param, log_dt):
    B, T, D = u.shape
    N = log_A.shape[0]
    bg = min(_BG, B)
    L = min(_L, T)
    n_g = B // bg
    grid = (n_g, T // L)
    h0r = h0.reshape(n_g, bg, N)
    return pl.pallas_call(
        _ssm_body,
        grid=grid,
        in_specs=[
            pl.BlockSpec((bg, L, D), lambda g, t: (g, t, 0)),
            pl.BlockSpec((1, bg, N), lambda g, t: (g, 0, 0)),
            pl.BlockSpec((1, N), lambda g, t: (0, 0)),
            pl.BlockSpec((1, N), lambda g, t: (0, 0)),
            pl.BlockSpec((1, N), lambda g, t: (0, 0)),
            pl.BlockSpec((D, N), lambda g, t: (0, 0)),
            pl.BlockSpec((N, D), lambda g, t: (0, 0)),
            pl.BlockSpec((1, D), lambda g, t: (0, 0)),
            pl.BlockSpec((1, D), lambda g, t: (0, 0)),
        ],
        out_specs=pl.BlockSpec((bg, L, D), lambda g, t: (g, t, 0)),
        out_shape=jax.ShapeDtypeStruct((B, T, D), jnp.float32),
        scratch_shapes=[pltpu.VMEM((8, N), jnp.float32)],
        compiler_params=pltpu.CompilerParams(
            dimension_semantics=("parallel", "arbitrary"),
            vmem_limit_bytes=56 * 1024 * 1024,
        ),
        name="ssm_layer",
    )(u, h0r, log_A.reshape(1, N), log_dt.reshape(1, N), B_b.reshape(1, N),
      B_w.T, C_w.T, C_b.reshape(1, D), D_param.reshape(1, D))
